# baseline (speedup 1.0000x reference)
import functools

import jax
import jax.numpy as jnp
from jax import lax
from jax.experimental import pallas as pl
from jax.experimental.pallas import tpu as pltpu
from jax.experimental.pallas import tpu_sc as plsc

_CH = 128
_NW = 32
_RAY_MIN = 0.0
_RAY_MAX = 1.0
_TS = 1024


def _make_sc_transpose(channels, dims):
    d0, d1, d2, d3 = dims
    n_cells = d0 * d1 * d2 * d3
    per_w = n_cells // _NW
    nslab = per_w // _TS
    mesh = plsc.VectorSubcoreMesh(core_axis_name="c", subcore_axis_name="s")

    @functools.partial(
        pl.kernel,
        out_type=jax.ShapeDtypeStruct((n_cells, channels), jnp.float32),
        mesh=mesh,
        compiler_params=pltpu.CompilerParams(
            needs_layout_passes=False, use_tc_tiling_on_sc=False
        ),
        scratch_types=[
            pltpu.VMEM((channels, _TS), jnp.float32),
            pltpu.VMEM((channels, _TS), jnp.float32),
            pltpu.VMEM((_TS, channels), jnp.float32),
            pltpu.VMEM((_TS, channels), jnp.float32),
            pltpu.SemaphoreType.DMA,
            pltpu.SemaphoreType.DMA,
            pltpu.SemaphoreType.DMA,
            pltpu.SemaphoreType.DMA,
        ],
    )
    def transpose(grid_hbm, table_hbm, in0, in1, out0, out1,
                  si0, si1, so0, so1):
        nc = 2
        wid = lax.axis_index("s") * nc + lax.axis_index("c")
        pbase0 = wid * per_w
        inp = (in0, in1)
        outp = (out0, out1)
        sis = (si0, si1)
        sos = (so0, so1)
        iota = jnp.arange(16, dtype=jnp.int32)
        cols = [jnp.full((16,), c, dtype=jnp.int32) for c in range(channels)]

        def fire_in(s, par):
            pltpu.async_copy(
                grid_hbm.at[:, pl.ds(pbase0 + s * _TS, _TS)],
                inp[par], sis[par],
            )

        def wait_in(par):
            pltpu.make_async_copy(
                grid_hbm.at[:, pl.ds(0, _TS)], inp[par], sis[par]
            ).wait()

        def compute(par):
            def cj(j, carry):
                rows_idx = iota + j * 16
                for c in range(channels):
                    v = inp[par][c, pl.ds(j * 16, 16)]
                    plsc.store_scatter(outp[par], [rows_idx, cols[c]], v)
                return carry

            lax.fori_loop(0, _TS // 16, cj, 0)

        def fire_out(s, par):
            pltpu.async_copy(
                outp[par],
                table_hbm.at[pl.ds(pbase0 + s * _TS, _TS)],
                sos[par],
            )

        def wait_out(par):
            pltpu.make_async_copy(
                table_hbm.at[pl.ds(0, _TS)], outp[par], sos[par]
            ).wait()

        fire_in(0, 0)

        def body(i2, carry):
            s0 = 2 * i2
            fire_in(s0 + 1, 1)
            wait_in(0)

            @pl.when(i2 > 0)
            def _():
                wait_out(0)

            compute(0)
            fire_out(s0, 0)

            @pl.when(i2 < nslab // 2 - 1)
            def _():
                fire_in(s0 + 2, 0)

            wait_in(1)

            @pl.when(i2 > 0)
            def _():
                wait_out(1)

            compute(1)
            fire_out(s0 + 1, 1)
            return carry

        lax.fori_loop(0, nslab // 2, body, 0)
        wait_out(0)
        wait_out(1)

    return transpose


def _make_sc_interp(n_rays, channels, dims):
    per_w = n_rays // _NW
    n_chunks = per_w // _CH
    d0, d1, d2, d3 = dims
    strides = (d1 * d2 * d3, d2 * d3, d3, 1)
    consts = []
    for k in range(16):
        bits = ((k >> 0) & 1, (k >> 1) & 1, (k >> 2) & 1, (k >> 3) & 1)
        consts.append(sum(b * s for b, s in zip(bits, strides)))

    mesh = plsc.VectorSubcoreMesh(core_axis_name="c", subcore_axis_name="s")

    @functools.partial(
        pl.kernel,
        out_type=jax.ShapeDtypeStruct((n_rays * channels,), jnp.float32),
        mesh=mesh,
        compiler_params=pltpu.CompilerParams(
            needs_layout_passes=False, use_tc_tiling_on_sc=False
        ),
        scratch_types=[
            pltpu.VMEM((2, 4, _CH), jnp.float32),
            pltpu.VMEM((2, 16, _CH), jnp.int32),
            pltpu.VMEM((2, 16, _CH), jnp.float32),
            pltpu.VMEM((16 * _CH, channels), jnp.float32),
            pltpu.VMEM((16 * _CH, channels), jnp.float32),
            pltpu.VMEM((_CH * channels,), jnp.float32),
            pltpu.VMEM((_CH * channels,), jnp.float32),
            pltpu.SemaphoreType.DMA,
            pltpu.SemaphoreType.DMA,
            pltpu.SemaphoreType.DMA,
            pltpu.SemaphoreType.DMA,
        ],
    )
    def interp(rayt_hbm, table_hbm, out_hbm, rayv, idxbuf, wbuf,
               rows0, rows1, outv0, outv1, sg0, sg1, so0, so1):
        nc = 2
        wid = lax.axis_index("s") * nc + lax.axis_index("c")
        base0 = wid * per_w
        rowsp = (rows0, rows1)
        outvp = (outv0, outv1)
        sgs = (sg0, sg1)
        sos = (so0, so1)
        iota = jnp.arange(16, dtype=jnp.int32)
        cols = [jnp.full((16,), c, dtype=jnp.int32) for c in range(channels)]

        def weights_phase(ci, par):
            rbase = base0 + ci * _CH
            pltpu.sync_copy(rayt_hbm.at[:, pl.ds(rbase, _CH)], rayv.at[par])

            def wg(g, carry):
                sl = pl.ds(g * 16, 16)
                ws = []
                bs = []
                for d in range(4):
                    x = rayv[par, d, sl]
                    ind = (x - _RAY_MIN) / (_RAY_MAX - _RAY_MIN) * 2.0 - 1.0
                    gi = (ind + 1.0) * 0.5 * float(dims[d] - 1)
                    b = jnp.minimum(
                        jnp.maximum(gi.astype(jnp.int32), 0), dims[d] - 2
                    )
                    ws.append(gi - b.astype(jnp.float32))
                    bs.append(b)
                basei = ((bs[0] * d1 + bs[1]) * d2 + bs[2]) * d3 + bs[3]
                t = [[1.0 - w, w] for w in ws]
                for k in range(16):
                    wt = t[0][(k >> 0) & 1] * t[1][(k >> 1) & 1]
                    wt = wt * t[2][(k >> 2) & 1]
                    wt = wt * t[3][(k >> 3) & 1]
                    wbuf[par, k, sl] = wt
                    idxbuf[par, k, sl] = basei + consts[k]
                return carry

            lax.fori_loop(0, _CH // 16, wg, 0)

        def fire_gathers(par):
            for k in range(16):
                pltpu.async_copy(
                    table_hbm.at[idxbuf.at[par, k]],
                    rowsp[par].at[pl.ds(k * _CH, _CH)],
                    sgs[par],
                )

        def wait_gathers(par):
            pltpu.make_async_copy(
                table_hbm.at[pl.ds(0, 16 * _CH)],
                rowsp[par],
                sgs[par],
            ).wait()

        def accum_phase(ci, par):
            def ag(g, carry):
                gbase = g * 16
                accs = None
                for k in range(16):
                    w_k = wbuf[par, k, pl.ds(gbase, 16)]
                    rk = iota + (gbase + k * _CH)
                    vals = [
                        w_k * plsc.load_gather(rowsp[par], [rk, cols[c]])
                        for c in range(channels)
                    ]
                    if accs is None:
                        accs = vals
                    else:
                        accs = [a + v for a, v in zip(accs, vals)]
                rout = iota * channels + gbase * channels
                for c in range(channels):
                    plsc.store_scatter(outvp[par], [rout + c], accs[c])
                return carry

            lax.fori_loop(0, _CH // 16, ag, 0, unroll=2)
            rbase = base0 + ci * _CH
            pltpu.async_copy(
                outvp[par],
                out_hbm.at[pl.ds(rbase * channels, _CH * channels)],
                sos[par],
            )

        def wait_out(par):
            pltpu.make_async_copy(
                out_hbm.at[pl.ds(0, _CH * channels)], outvp[par], sos[par]
            ).wait()

        weights_phase(0, 0)
        fire_gathers(0)

        def body(i2, carry):
            c0 = 2 * i2
            weights_phase(c0 + 1, 1)
            fire_gathers(1)
            wait_gathers(0)

            @pl.when(i2 > 0)
            def _():
                wait_out(0)

            accum_phase(c0, 0)

            @pl.when(i2 < n_chunks // 2 - 1)
            def _():
                weights_phase(c0 + 2, 0)
                fire_gathers(0)

            wait_gathers(1)

            @pl.when(i2 > 0)
            def _():
                wait_out(1)

            accum_phase(c0 + 1, 1)
            return carry

        lax.fori_loop(0, n_chunks // 2, body, 0)
        wait_out(0)
        wait_out(1)

    return interp


def kernel(ray, grid):
    n_rays = ray.shape[0]
    channels = grid.shape[0]
    dims = grid.shape[1:]
    transpose = _make_sc_transpose(channels, dims)
    table = transpose(grid.reshape(channels, -1))
    rayt = ray.T
    interp = _make_sc_interp(n_rays, channels, dims)
    return interp(rayt, table).reshape(n_rays, channels)


# speedup vs baseline: 1.0396x; 1.0396x over previous
import functools

import jax
import jax.numpy as jnp
from jax import lax
from jax.experimental import pallas as pl
from jax.experimental.pallas import tpu as pltpu
from jax.experimental.pallas import tpu_sc as plsc

_CH = 128
_NW = 32
_RAY_MIN = 0.0
_RAY_MAX = 1.0
_TS = 1024


def _make_sc_transpose(channels, dims):
    d0, d1, d2, d3 = dims
    n_cells = d0 * d1 * d2 * d3
    per_w = n_cells // _NW
    nslab = per_w // _TS
    mesh = plsc.VectorSubcoreMesh(core_axis_name="c", subcore_axis_name="s")

    @functools.partial(
        pl.kernel,
        out_type=jax.ShapeDtypeStruct((n_cells, channels), jnp.float32),
        mesh=mesh,
        compiler_params=pltpu.CompilerParams(
            needs_layout_passes=False, use_tc_tiling_on_sc=False
        ),
        scratch_types=[
            pltpu.VMEM((channels, d2, d3), jnp.float32),
            pltpu.VMEM((channels, d2, d3), jnp.float32),
            pltpu.VMEM((_TS, channels), jnp.float32),
            pltpu.VMEM((_TS, channels), jnp.float32),
            pltpu.SemaphoreType.DMA,
            pltpu.SemaphoreType.DMA,
            pltpu.SemaphoreType.DMA,
            pltpu.SemaphoreType.DMA,
        ],
    )
    def transpose(grid_hbm, table_hbm, in0, in1, out0, out1,
                  si0, si1, so0, so1):
        nc = 2
        wid = lax.axis_index("s") * nc + lax.axis_index("c")
        pbase0 = wid * per_w
        inp = (in0, in1)
        outp = (out0, out1)
        sis = (si0, si1)
        sos = (so0, so1)
        iota = jnp.arange(16, dtype=jnp.int32)
        cols = [jnp.full((16,), c, dtype=jnp.int32) for c in range(channels)]

        def fire_in(s, par):
            pltpu.async_copy(grid_hbm.at[:, wid, s], inp[par], sis[par])

        def wait_in(par):
            pltpu.make_async_copy(
                grid_hbm.at[:, 0, 0], inp[par], sis[par]
            ).wait()

        def compute(par):
            def cj(r, carry):
                for h in range(d3 // 16):
                    rows_idx = iota + (r * d3 + h * 16)
                    for c in range(channels):
                        v = inp[par][c, r, pl.ds(h * 16, 16)]
                        plsc.store_scatter(
                            outp[par], [rows_idx, cols[c]], v
                        )
                return carry

            lax.fori_loop(0, d2, cj, 0)

        def fire_out(s, par):
            pltpu.async_copy(
                outp[par],
                table_hbm.at[pl.ds(pbase0 + s * _TS, _TS)],
                sos[par],
            )

        def wait_out(par):
            pltpu.make_async_copy(
                table_hbm.at[pl.ds(0, _TS)], outp[par], sos[par]
            ).wait()

        fire_in(0, 0)

        def body(i2, carry):
            s0 = 2 * i2
            fire_in(s0 + 1, 1)
            wait_in(0)

            @pl.when(i2 > 0)
            def _():
                wait_out(0)

            compute(0)
            fire_out(s0, 0)

            @pl.when(i2 < nslab // 2 - 1)
            def _():
                fire_in(s0 + 2, 0)

            wait_in(1)

            @pl.when(i2 > 0)
            def _():
                wait_out(1)

            compute(1)
            fire_out(s0 + 1, 1)
            return carry

        lax.fori_loop(0, nslab // 2, body, 0)
        wait_out(0)
        wait_out(1)

    return transpose


def _make_sc_interp(n_rays, channels, dims):
    per_w = n_rays // _NW
    n_chunks = per_w // _CH
    d0, d1, d2, d3 = dims
    strides = (d1 * d2 * d3, d2 * d3, d3, 1)
    consts = []
    for k in range(16):
        bits = ((k >> 0) & 1, (k >> 1) & 1, (k >> 2) & 1, (k >> 3) & 1)
        consts.append(sum(b * s for b, s in zip(bits, strides)))

    mesh = plsc.VectorSubcoreMesh(core_axis_name="c", subcore_axis_name="s")

    @functools.partial(
        pl.kernel,
        out_type=jax.ShapeDtypeStruct((n_rays * channels,), jnp.float32),
        mesh=mesh,
        compiler_params=pltpu.CompilerParams(
            needs_layout_passes=False, use_tc_tiling_on_sc=False
        ),
        scratch_types=[
            pltpu.VMEM((2, 4, _CH), jnp.float32),
            pltpu.VMEM((2, 16, _CH), jnp.int32),
            pltpu.VMEM((2, 16, _CH), jnp.float32),
            pltpu.VMEM((16 * _CH, channels), jnp.float32),
            pltpu.VMEM((16 * _CH, channels), jnp.float32),
            pltpu.VMEM((_CH * channels,), jnp.float32),
            pltpu.VMEM((_CH * channels,), jnp.float32),
            pltpu.SemaphoreType.DMA,
            pltpu.SemaphoreType.DMA,
            pltpu.SemaphoreType.DMA,
            pltpu.SemaphoreType.DMA,
        ],
    )
    def interp(rayt_hbm, table_hbm, out_hbm, rayv, idxbuf, wbuf,
               rows0, rows1, outv0, outv1, sg0, sg1, so0, so1):
        nc = 2
        wid = lax.axis_index("s") * nc + lax.axis_index("c")
        base0 = wid * per_w
        rowsp = (rows0, rows1)
        outvp = (outv0, outv1)
        sgs = (sg0, sg1)
        sos = (so0, so1)
        iota = jnp.arange(16, dtype=jnp.int32)
        cols = [jnp.full((16,), c, dtype=jnp.int32) for c in range(channels)]

        def weights_phase(ci, par):
            rbase = base0 + ci * _CH
            pltpu.sync_copy(rayt_hbm.at[:, pl.ds(rbase, _CH)], rayv.at[par])

            def wg(g, carry):
                sl = pl.ds(g * 16, 16)
                ws = []
                bs = []
                for d in range(4):
                    x = rayv[par, d, sl]
                    ind = (x - _RAY_MIN) / (_RAY_MAX - _RAY_MIN) * 2.0 - 1.0
                    gi = (ind + 1.0) * 0.5 * float(dims[d] - 1)
                    b = jnp.minimum(
                        jnp.maximum(gi.astype(jnp.int32), 0), dims[d] - 2
                    )
                    ws.append(gi - b.astype(jnp.float32))
                    bs.append(b)
                basei = ((bs[0] * d1 + bs[1]) * d2 + bs[2]) * d3 + bs[3]
                t = [[1.0 - w, w] for w in ws]
                for k in range(16):
                    wt = t[0][(k >> 0) & 1] * t[1][(k >> 1) & 1]
                    wt = wt * t[2][(k >> 2) & 1]
                    wt = wt * t[3][(k >> 3) & 1]
                    wbuf[par, k, sl] = wt
                    idxbuf[par, k, sl] = basei + consts[k]
                return carry

            lax.fori_loop(0, _CH // 16, wg, 0)

        def fire_gathers(par):
            for k in range(16):
                pltpu.async_copy(
                    table_hbm.at[idxbuf.at[par, k]],
                    rowsp[par].at[pl.ds(k * _CH, _CH)],
                    sgs[par],
                )

        def wait_gathers(par):
            pltpu.make_async_copy(
                table_hbm.at[pl.ds(0, 16 * _CH)],
                rowsp[par],
                sgs[par],
            ).wait()

        def accum_phase(ci, par):
            def ag(g, carry):
                gbase = g * 16
                accs = None
                for k in range(16):
                    w_k = wbuf[par, k, pl.ds(gbase, 16)]
                    rk = iota + (gbase + k * _CH)
                    vals = [
                        w_k * plsc.load_gather(rowsp[par], [rk, cols[c]])
                        for c in range(channels)
                    ]
                    if accs is None:
                        accs = vals
                    else:
                        accs = [a + v for a, v in zip(accs, vals)]
                rout = iota * channels + gbase * channels
                for c in range(channels):
                    plsc.store_scatter(outvp[par], [rout + c], accs[c])
                return carry

            lax.fori_loop(0, _CH // 16, ag, 0)
            rbase = base0 + ci * _CH
            pltpu.async_copy(
                outvp[par],
                out_hbm.at[pl.ds(rbase * channels, _CH * channels)],
                sos[par],
            )

        def wait_out(par):
            pltpu.make_async_copy(
                out_hbm.at[pl.ds(0, _CH * channels)], outvp[par], sos[par]
            ).wait()

        weights_phase(0, 0)
        fire_gathers(0)

        def body(i2, carry):
            c0 = 2 * i2
            weights_phase(c0 + 1, 1)
            fire_gathers(1)
            wait_gathers(0)

            @pl.when(i2 > 0)
            def _():
                wait_out(0)

            accum_phase(c0, 0)

            @pl.when(i2 < n_chunks // 2 - 1)
            def _():
                weights_phase(c0 + 2, 0)
                fire_gathers(0)

            wait_gathers(1)

            @pl.when(i2 > 0)
            def _():
                wait_out(1)

            accum_phase(c0 + 1, 1)
            return carry

        lax.fori_loop(0, n_chunks // 2, body, 0)
        wait_out(0)
        wait_out(1)

    return interp


def kernel(ray, grid):
    n_rays = ray.shape[0]
    channels = grid.shape[0]
    dims = grid.shape[1:]
    transpose = _make_sc_transpose(channels, dims)
    table = transpose(grid)
    rayt = ray.T
    interp = _make_sc_interp(n_rays, channels, dims)
    return interp(rayt, table).reshape(n_rays, channels)
